# baseline (device time: 12718 ns/iter reference)
import jax
import jax.numpy as jnp
from jax import lax
from jax.experimental import pallas as pl
from jax.experimental.pallas import tpu as pltpu


def kernel(x):
    m, n = x.shape
    h = m // 2

    def body(x_ref, out_ref, comm_ref, send_sems, recv_sems):
        my_x = lax.axis_index("x")
        my_y = lax.axis_index("y")
        other_x = 1 - my_x
        other_y = 1 - my_y

        barrier_sem = pltpu.get_barrier_semaphore()
        pl.semaphore_signal(
            barrier_sem,
            inc=1,
            device_id=(other_x, my_y),
            device_id_type=pl.DeviceIdType.MESH,
        )
        pl.semaphore_signal(
            barrier_sem,
            inc=1,
            device_id=(other_x, other_y),
            device_id_type=pl.DeviceIdType.MESH,
        )
        pl.semaphore_wait(barrier_sem, 2)

        comm_ref[0] = x_ref[...].astype(jnp.bfloat16)

        rdma_near = pltpu.make_async_remote_copy(
            src_ref=comm_ref.at[0, pl.ds(0, h)],
            dst_ref=comm_ref.at[1, pl.ds(0, h)],
            send_sem=send_sems.at[0],
            recv_sem=recv_sems.at[0],
            device_id=(other_x, my_y),
            device_id_type=pl.DeviceIdType.MESH,
        )
        rdma_diag = pltpu.make_async_remote_copy(
            src_ref=comm_ref.at[0, pl.ds(h, h)],
            dst_ref=comm_ref.at[1, pl.ds(h, h)],
            send_sem=send_sems.at[1],
            recv_sem=recv_sems.at[1],
            device_id=(other_x, other_y),
            device_id_type=pl.DeviceIdType.MESH,
        )
        rdma_near.start()
        rdma_diag.start()

        out_ref[pl.ds(my_x * m, m), :] = comm_ref[0]

        rdma_near.wait()
        rdma_diag.wait()
        out_ref[pl.ds(other_x * m, m), :] = comm_ref[1]

    return pl.pallas_call(
        body,
        out_shape=jax.ShapeDtypeStruct((2 * m, n), jnp.bfloat16),
        in_specs=[pl.BlockSpec(memory_space=pltpu.VMEM)],
        out_specs=pl.BlockSpec(memory_space=pltpu.VMEM),
        scratch_shapes=[
            pltpu.VMEM((2, m, n), jnp.bfloat16),
            pltpu.SemaphoreType.DMA((2,)),
            pltpu.SemaphoreType.DMA((2,)),
        ],
        compiler_params=pltpu.CompilerParams(collective_id=0),
    )(x)


# device time: 11938 ns/iter; 1.0653x vs baseline; 1.0653x over previous
import jax
import jax.numpy as jnp
from jax import lax
from jax.experimental import pallas as pl
from jax.experimental.pallas import tpu as pltpu

P = 4


def kernel(x):
    m, n = x.shape
    q = m // 2
    s = q // P

    def body(x_ref, out_ref, send_buf, dq_buf, fq_buf,
             x_send_sems, x_recv_sems, y_send_sems, y_recv_sems):
        my_x = lax.axis_index("x")
        my_y = lax.axis_index("y")
        other_x = 1 - my_x
        other_y = 1 - my_y

        barrier_sem = pltpu.get_barrier_semaphore()
        pl.semaphore_signal(
            barrier_sem, inc=1,
            device_id=(other_x, my_y), device_id_type=pl.DeviceIdType.MESH,
        )
        pl.semaphore_signal(
            barrier_sem, inc=1,
            device_id=(my_x, other_y), device_id_type=pl.DeviceIdType.MESH,
        )

        send_buf[...] = x_ref[pl.ds(my_y * q, q), :].astype(jnp.bfloat16)
        out_ref[pl.ds(my_x * m, m), :] = x_ref[...].astype(jnp.bfloat16)

        pl.semaphore_wait(barrier_sem, 2)

        x_rdmas = []
        for p in range(P):
            rdma = pltpu.make_async_remote_copy(
                src_ref=send_buf.at[pl.ds(p * s, s)],
                dst_ref=dq_buf.at[pl.ds(p * s, s)],
                send_sem=x_send_sems.at[p],
                recv_sem=x_recv_sems.at[p],
                device_id=(other_x, my_y),
                device_id_type=pl.DeviceIdType.MESH,
            )
            rdma.start()
            x_rdmas.append(rdma)

        y_rdmas = []
        for p in range(P):
            x_rdmas[p].wait()
            rdma = pltpu.make_async_remote_copy(
                src_ref=dq_buf.at[pl.ds(p * s, s)],
                dst_ref=fq_buf.at[pl.ds(p * s, s)],
                send_sem=y_send_sems.at[p],
                recv_sem=y_recv_sems.at[p],
                device_id=(my_x, other_y),
                device_id_type=pl.DeviceIdType.MESH,
            )
            rdma.start()
            y_rdmas.append(rdma)
            out_ref[pl.ds(other_x * m + my_y * q + p * s, s), :] = \
                dq_buf[pl.ds(p * s, s), :]

        for p in range(P):
            y_rdmas[p].wait()
            out_ref[pl.ds(other_x * m + other_y * q + p * s, s), :] = \
                fq_buf[pl.ds(p * s, s), :]

    return pl.pallas_call(
        body,
        out_shape=jax.ShapeDtypeStruct((2 * m, n), jnp.bfloat16),
        in_specs=[pl.BlockSpec(memory_space=pltpu.VMEM)],
        out_specs=pl.BlockSpec(memory_space=pltpu.VMEM),
        scratch_shapes=[
            pltpu.VMEM((q, n), jnp.bfloat16),
            pltpu.VMEM((q, n), jnp.bfloat16),
            pltpu.VMEM((q, n), jnp.bfloat16),
            pltpu.SemaphoreType.DMA((P,)),
            pltpu.SemaphoreType.DMA((P,)),
            pltpu.SemaphoreType.DMA((P,)),
            pltpu.SemaphoreType.DMA((P,)),
        ],
        compiler_params=pltpu.CompilerParams(collective_id=0),
    )(x)


# device time: 11108 ns/iter; 1.1449x vs baseline; 1.0747x over previous
import jax
import jax.numpy as jnp
from jax import lax
from jax.experimental import pallas as pl
from jax.experimental.pallas import tpu as pltpu

FWD = 160
FP = 4
FS = FWD // FP
KEEP = 192
KP = 2
KS = KEEP // KP
KBASE = 160


def kernel(x):
    m, n = x.shape
    assert FWD + KEEP + FWD == m

    def body(x_ref, out_ref, xb, other_buf,
             xf_send, xf_recv, xk_send, xk_recv, yf_send, yf_recv):
        my_x = lax.axis_index("x")
        my_y = lax.axis_index("y")
        other_x = 1 - my_x
        other_y = 1 - my_y
        fb = my_y * (KBASE + KEEP)
        ifb = other_y * (KBASE + KEEP)
        obase = other_x * m

        barrier_sem = pltpu.get_barrier_semaphore()
        pl.semaphore_signal(
            barrier_sem, inc=1,
            device_id=(other_x, my_y), device_id_type=pl.DeviceIdType.MESH,
        )
        pl.semaphore_signal(
            barrier_sem, inc=1,
            device_id=(my_x, other_y), device_id_type=pl.DeviceIdType.MESH,
        )

        xb[...] = x_ref[...].astype(jnp.bfloat16)
        out_ref[pl.ds(my_x * m, m), :] = xb[...]

        pl.semaphore_wait(barrier_sem, 2)

        xf_rdmas = []
        for p in range(FP):
            rdma = pltpu.make_async_remote_copy(
                src_ref=xb.at[pl.ds(fb + p * FS, FS)],
                dst_ref=other_buf.at[pl.ds(fb + p * FS, FS)],
                send_sem=xf_send.at[p],
                recv_sem=xf_recv.at[p],
                device_id=(other_x, my_y),
                device_id_type=pl.DeviceIdType.MESH,
            )
            rdma.start()
            xf_rdmas.append(rdma)
        xk_rdmas = []
        for p in range(KP):
            rdma = pltpu.make_async_remote_copy(
                src_ref=xb.at[pl.ds(KBASE + p * KS, KS)],
                dst_ref=other_buf.at[pl.ds(KBASE + p * KS, KS)],
                send_sem=xk_send.at[p],
                recv_sem=xk_recv.at[p],
                device_id=(other_x, my_y),
                device_id_type=pl.DeviceIdType.MESH,
            )
            rdma.start()
            xk_rdmas.append(rdma)

        yf_rdmas = []
        for p in range(FP):
            xf_rdmas[p].wait()
            rdma = pltpu.make_async_remote_copy(
                src_ref=other_buf.at[pl.ds(fb + p * FS, FS)],
                dst_ref=other_buf.at[pl.ds(fb + p * FS, FS)],
                send_sem=yf_send.at[p],
                recv_sem=yf_recv.at[p],
                device_id=(my_x, other_y),
                device_id_type=pl.DeviceIdType.MESH,
            )
            rdma.start()
            yf_rdmas.append(rdma)
            out_ref[pl.ds(obase + fb + p * FS, FS), :] = \
                other_buf[pl.ds(fb + p * FS, FS), :]

        for p in range(KP):
            xk_rdmas[p].wait()
            out_ref[pl.ds(obase + KBASE + p * KS, KS), :] = \
                other_buf[pl.ds(KBASE + p * KS, KS), :]

        for p in range(FP):
            yf_rdmas[p].wait()
            out_ref[pl.ds(obase + ifb + p * FS, FS), :] = \
                other_buf[pl.ds(ifb + p * FS, FS), :]

    return pl.pallas_call(
        body,
        out_shape=jax.ShapeDtypeStruct((2 * m, n), jnp.bfloat16),
        in_specs=[pl.BlockSpec(memory_space=pltpu.VMEM)],
        out_specs=pl.BlockSpec(memory_space=pltpu.VMEM),
        scratch_shapes=[
            pltpu.VMEM((m, n), jnp.bfloat16),
            pltpu.VMEM((m, n), jnp.bfloat16),
            pltpu.SemaphoreType.DMA((FP,)),
            pltpu.SemaphoreType.DMA((FP,)),
            pltpu.SemaphoreType.DMA((KP,)),
            pltpu.SemaphoreType.DMA((KP,)),
            pltpu.SemaphoreType.DMA((FP,)),
            pltpu.SemaphoreType.DMA((FP,)),
        ],
        compiler_params=pltpu.CompilerParams(collective_id=0),
    )(x)
